# baseline (speedup 1.0000x reference)
import functools

import jax
import jax.numpy as jnp
from jax import lax
from jax.experimental import pallas as pl
from jax.experimental.pallas import tpu as pltpu
from jax.experimental.pallas import tpu_sc as plsc

_NC = 2
_NS = 16
_NW = _NC * _NS
_LANES = 16


def _leaky(x):
    return jnp.maximum(x, 0.1 * x)


def _sc_layer(tables, idx, wp, M, K, NB, chunk_src, xyz_src, Co):
    P = M // _NW
    RB = NB * K
    nh = RB // 128
    assert RB % 128 == 0
    nblk = P // NB
    assert nblk % 2 == 0
    nc = Co // _LANES
    nt = len(tables)
    Rs = [t.shape[1] for t in tables]

    mesh = plsc.VectorSubcoreMesh(core_axis_name="c", subcore_axis_name="s")

    @functools.partial(
        pl.kernel,
        mesh=mesh,
        compiler_params=pltpu.CompilerParams(use_tc_tiling_on_sc=False),
        out_type=jax.ShapeDtypeStruct((M, Co), jnp.float32),
        scratch_types=[
            pltpu.VMEM((P * K,), jnp.int32),
            pltpu.VMEM((P, 16), jnp.float32),
            pltpu.VMEM((2, NB, Co), jnp.float32),
            pltpu.VMEM((4, Co), jnp.float32),
        ] + [pltpu.VMEM((2, RB, R), jnp.float32) for R in Rs] + [
            pltpu.SemaphoreType.DMA,
            pltpu.SemaphoreType.DMA,
            pltpu.SemaphoreType.DMA,
            pltpu.SemaphoreType.DMA,
        ],
    )
    def k(*refs):
        tabs_hbm = refs[:nt]
        idx_hbm, w_hbm, agg_hbm, idx_v, cen_v, out_v, w_v = refs[nt:nt + 7]
        rows_vs = refs[nt + 7:nt + 7 + nt]
        sg0, sg1, so0, so1 = refs[nt + 7 + nt:]
        wid = lax.axis_index("s") * _NC + lax.axis_index("c")
        base = wid * P
        pltpu.sync_copy(w_hbm, w_v)
        pltpu.sync_copy(idx_hbm.at[pl.ds(base * K, P * K)], idx_v)
        cti, cto = xyz_src
        pltpu.sync_copy(
            tabs_hbm[cti].at[pl.ds(base, P), pl.ds(cto, 16)], cen_v)
        sg = [sg0, sg1]
        so = [so0, so1]
        w0 = [w_v[0, pl.ds(c * 16, 16)] for c in range(nc)]
        w1 = [w_v[1, pl.ds(c * 16, 16)] for c in range(nc)]
        w2 = [w_v[2, pl.ds(c * 16, 16)] for c in range(nc)]
        wb = [w_v[3, pl.ds(c * 16, 16)] for c in range(nc)]

        def gather_parts(t, buf, s):
            return [
                (tabs_hbm[ti].at[idx_v.at[pl.ds(t * RB + h * 128, 128)]],
                 rows_vs[ti].at[buf, pl.ds(h * 128, 128)], s)
                for ti in range(nt)
                for h in range(nh)
            ]

        def start_gather(t, buf, s):
            for src, dst, sm in gather_parts(t, buf, s):
                pltpu.async_copy(src, dst, sm)

        def wait_gather(t, buf, s):
            for src, dst, sm in gather_parts(t, buf, s):
                pltpu.make_async_copy(src, dst, sm).wait()

        start_gather(0, 0, sg[0])

        def tt_body(tt, _):
            for b in range(2):
                t = tt * 2 + b

                @pl.when(t + 1 < nblk)
                def _():
                    start_gather(t + 1, 1 - b, sg[1 - b])

                wait_gather(t, b, sg[b])

                @pl.when(t >= 2)
                def _():
                    pltpu.make_async_copy(
                        out_v.at[b], agg_hbm.at[pl.ds(base, NB)],
                        so[b]).wait()

                xti, xto = xyz_src

                def point(i, _):
                    cen = cen_v[t * NB + i, :]
                    accs = [jnp.zeros((16,), jnp.float32)
                            for _ in range(nc)]
                    for j in range(K):
                        r = i * K + j
                        diff = rows_vs[xti][b, r, pl.ds(xto, 16)] - cen
                        rx = diff[0]
                        ry = diff[1]
                        rz = diff[2]
                        for c in range(nc):
                            t0 = (rx * w0[c] + ry * w1[c] + rz * w2[c]
                                  + wb[c])
                            w = _leaky(t0)
                            fti, fto = chunk_src[c]
                            f = rows_vs[fti][b, r, pl.ds(fto, 16)]
                            accs[c] = accs[c] + w * f
                    for c in range(nc):
                        out_v[b, i, pl.ds(c * 16, 16)] = accs[c]
                    return 0

                lax.fori_loop(0, NB, point, 0)
                pltpu.async_copy(
                    out_v.at[b], agg_hbm.at[pl.ds(base + t * NB, NB)],
                    so[b])
            return 0

        lax.fori_loop(0, nblk // 2, tt_body, 0)
        for b in range(2):
            pltpu.make_async_copy(
                out_v.at[b], agg_hbm.at[pl.ds(base, NB)], so[b]).wait()

    return k(*tables, idx, wp)


def _tc_linear(x, wT, b, x16=None):
    M, C = x.shape
    O = wT.shape[1]
    BM = 2048
    E = 16 if x16 is not None else 0

    def body(*refs):
        if E:
            x_ref, w_ref, b_ref, e_ref, o_ref = refs
        else:
            x_ref, w_ref, b_ref, o_ref = refs
        y = jnp.dot(x_ref[...], w_ref[...],
                    preferred_element_type=jnp.float32,
                    precision=lax.Precision.HIGHEST)
        y = y + b_ref[...]
        o_ref[:, :O] = jnp.maximum(y, 0.1 * y)
        if E:
            o_ref[:, O:] = e_ref[...]

    in_specs = [
        pl.BlockSpec((BM, C), lambda i: (i, 0)),
        pl.BlockSpec((C, O), lambda i: (0, 0)),
        pl.BlockSpec((1, O), lambda i: (0, 0)),
    ]
    args = [x, wT, b]
    if E:
        in_specs.append(pl.BlockSpec((BM, 16), lambda i: (i, 0)))
        args.append(x16)

    return pl.pallas_call(
        body,
        grid=(M // BM,),
        in_specs=in_specs,
        out_specs=pl.BlockSpec((BM, O + E), lambda i: (i, 0)),
        out_shape=jax.ShapeDtypeStruct((M, O + E), jnp.float32),
    )(*args)


def _tc_final(x, wT, b, flow, B, N):
    BM = 2048

    def body(x_ref, w_ref, b_ref, f_ref, o_ref):
        y = jnp.dot(x_ref[0], w_ref[...],
                    preferred_element_type=jnp.float32,
                    precision=lax.Precision.HIGHEST)
        y = y + b_ref[...]
        y = jnp.maximum(y, 0.1 * y)
        yt = jnp.swapaxes(y, 0, 1)
        o_ref[0] = jnp.concatenate([yt[:125, :], f_ref[0]], axis=0)

    return pl.pallas_call(
        body,
        grid=(B, N // BM),
        in_specs=[
            pl.BlockSpec((1, BM, 144), lambda bi, i: (bi, i, 0)),
            pl.BlockSpec((144, 128), lambda bi, i: (0, 0)),
            pl.BlockSpec((1, 128), lambda bi, i: (0, 0)),
            pl.BlockSpec((1, 3, BM), lambda bi, i: (bi, 0, i)),
        ],
        out_specs=pl.BlockSpec((1, 128, BM), lambda bi, i: (bi, 0, i)),
        out_shape=jax.ShapeDtypeStruct((B, 128, N), jnp.float32),
    )(x, wT, b, flow)


def _tc_transpose(corr, B, C, N):
    BM = 2048
    nb = N // BM

    def body(x_ref, o_ref):
        o_ref[...] = jnp.swapaxes(x_ref[0], 0, 1)

    return pl.pallas_call(
        body,
        grid=(B, nb),
        in_specs=[
            pl.BlockSpec((1, C, BM), lambda bi, i: (bi, 0, i)),
        ],
        out_specs=pl.BlockSpec((BM, C), lambda bi, i: (bi * nb + i, 0)),
        out_shape=jax.ShapeDtypeStruct((B * N, C), jnp.float32),
    )(corr)


def _wpack(Wwn, bwn, pad_to=None):
    wp = jnp.concatenate([Wwn.T, bwn[None, :]], axis=0)
    if pad_to is not None and wp.shape[1] < pad_to:
        wp = jnp.pad(wp, ((0, 0), (0, pad_to - wp.shape[1])))
    return wp.astype(jnp.float32)


def _linpack(Wlin, blin, cin_pad=None, o_pad=None):
    wT = Wlin.T
    C, O = wT.shape
    if cin_pad is not None and C < cin_pad:
        wT = jnp.pad(wT, ((0, cin_pad - C), (0, 0)))
    if o_pad is not None and O < o_pad:
        wT = jnp.pad(wT, ((0, 0), (0, o_pad - O)))
        blin = jnp.pad(blin, (0, o_pad - O))
    return wT.astype(jnp.float32), blin[None, :].astype(jnp.float32)


def kernel(xyz, flow, corr, knn_indices,
           Wwn_c1, bwn_c1, Wlin_c1, blin_c1,
           Wwn_f1, bwn_f1, Wlin_f1, blin_f1,
           Wwn_f2, bwn_f2, Wlin_f2, blin_f2,
           Wwn_o, bwn_o, Wlin_o, blin_o):
    B, _, N = xyz.shape
    M = B * N
    f32 = jnp.float32
    corr = corr.astype(f32)
    flow = flow.astype(f32)

    xyzT = jnp.swapaxes(xyz, 1, 2).reshape(M, 3)
    flowT = jnp.swapaxes(flow, 1, 2).reshape(M, 3)
    z13 = jnp.zeros((M, 13), f32)
    z5 = jnp.zeros((M, 5), f32)
    xyz16 = jnp.concatenate([xyzT, z13], axis=1)
    tab0 = jnp.concatenate(
        [xyzT, z5, flowT, z5], axis=1)
    offs = (jnp.arange(B, dtype=jnp.int32) * N)[:, None, None]
    idxg = knn_indices.astype(jnp.int32) + offs
    idx32 = idxg.reshape(M * 32)
    idx16 = idxg[:, :, :16].reshape(M * 16)
    corrT = _tc_transpose(corr, B, 128, N)

    wp_f1 = jnp.pad(_wpack(Wwn_f1, bwn_f1), ((0, 0), (8, 5)))
    agg_f1 = _sc_layer([tab0], idx32, wp_f1,
                       M, K=32, NB=8,
                       chunk_src=[(0, 0)], xyz_src=(0, 0), Co=16)
    wT = jnp.pad(Wlin_f1.T, ((8, 5), (0, 0))).astype(f32)
    bb = blin_f1[None, :].astype(f32)
    tab_f = _tc_linear(agg_f1, wT, bb, x16=xyz16)

    agg_c1 = _sc_layer([corrT, xyz16], idx16, _wpack(Wwn_c1, bwn_c1),
                       M, K=16, NB=16,
                       chunk_src=[(0, c * 16) for c in range(8)],
                       xyz_src=(1, 0), Co=128)
    wT, bb = _linpack(Wlin_c1, blin_c1)
    cf = _tc_linear(agg_c1, wT, bb)

    agg_f2 = _sc_layer([tab_f], idx16, _wpack(Wwn_f2, bwn_f2),
                       M, K=16, NB=16,
                       chunk_src=[(0, 0), (0, 16)], xyz_src=(0, 32), Co=32)
    wT, bb = _linpack(Wlin_f2, blin_f2)
    ff2x = _tc_linear(agg_f2, wT, bb, x16=xyz16)

    agg_o = _sc_layer([cf, ff2x], idx16, _wpack(Wwn_o, bwn_o),
                      M, K=16, NB=16,
                      chunk_src=[(0, c * 16) for c in range(8)] + [(1, 0)],
                      xyz_src=(1, 16), Co=144)
    wT, bb = _linpack(Wlin_o, blin_o, o_pad=128)
    return _tc_final(agg_o.reshape(B, N, 144), wT, bb, flow, B, N)
